# baseline (speedup 1.0000x reference)
import functools

import jax
import jax.numpy as jnp
from jax import lax
from jax.experimental import pallas as pl
from jax.experimental.pallas import tpu as pltpu
from jax.experimental.pallas import tpu_sc as plsc

NC = 2
NS = 16
K = 80


def _mesh():
    return plsc.VectorSubcoreMesh(core_axis_name="c", subcore_axis_name="s")


def _pad_rows(n):
    return ((n + NS * 8 - 1) // (NS * 8)) * NS * 8


def _sc_prop(table, src4, dst4, zeros, NP, W):
    nch = src4.shape[2]
    rpt = NP // NS
    nh1 = ((nch + 1) // 2 + 7) // 8 * 8
    halves = ((0, nh1), (nh1, nch - nh1))

    @functools.partial(
        pl.kernel,
        out_type=jax.ShapeDtypeStruct((NC, NP, W), jnp.float32),
        mesh=_mesh(),
        scratch_types=[
            pltpu.VMEM((nh1, K), jnp.int32),
            pltpu.VMEM((nh1, K), jnp.int32),
            pltpu.VMEM((2, K, W), jnp.float32),
            pltpu.VMEM_SHARED((NP, W), jnp.float32),
            pltpu.SemaphoreType.DMA,
        ],
    )
    def body(t_hbm, src_hbm, dst_hbm, z_hbm, out_hbm,
             src_v, dst_v, rows2, acc, sem):
        cid = lax.axis_index("c")
        sid = lax.axis_index("s")
        r0 = sid * rpt
        pltpu.sync_copy(z_hbm.at[pl.ds(r0, rpt)], acc.at[pl.ds(r0, rpt)])
        plsc.subcore_barrier()

        for base, cnt in halves:
            pltpu.sync_copy(src_hbm.at[cid, sid, pl.ds(base, cnt)],
                            src_v.at[pl.ds(0, cnt)])
            pltpu.sync_copy(dst_hbm.at[cid, sid, pl.ds(base, cnt)],
                            dst_v.at[pl.ds(0, cnt)])
            pltpu.async_copy(t_hbm.at[src_v.at[0]], rows2.at[0], sem)

            def step(i, carry):
                p = lax.rem(i, 2)
                pltpu.make_async_copy(
                    t_hbm.at[src_v.at[i]], rows2.at[p], sem).wait()

                @pl.when(i + 1 < cnt)
                def _():
                    pltpu.async_copy(
                        t_hbm.at[src_v.at[i + 1]], rows2.at[1 - p], sem)

                pltpu.sync_copy(rows2.at[p], acc.at[dst_v.at[i]], add=True)
                return carry

            lax.fori_loop(0, cnt, step, 0)

        plsc.subcore_barrier()
        pltpu.sync_copy(acc.at[pl.ds(r0, rpt)], out_hbm.at[cid, pl.ds(r0, rpt)])

    return body(table, src4, dst4, zeros)


def _tc_prep(cnt, x, N, C, R):
    def body(cnt_ref, x_ref, dinv_ref, y1_ref):
        deg = cnt_ref[0][:, :1] + cnt_ref[1][:, :1] + 1.0
        dinv = lax.rsqrt(deg)
        dinv_ref[...] = dinv
        y1 = x_ref[...] * dinv
        y1_ref[0] = y1
        y1_ref[1] = y1

    return pl.pallas_call(
        body,
        grid=(N // R,),
        in_specs=[
            pl.BlockSpec((NC, R, C), lambda i: (0, i, 0)),
            pl.BlockSpec((R, C), lambda i: (i, 0)),
        ],
        out_specs=[
            pl.BlockSpec((R, 1), lambda i: (i, 0)),
            pl.BlockSpec((NC, R, C), lambda i: (0, i, 0)),
        ],
        out_shape=[
            jax.ShapeDtypeStruct((N, 1), jnp.float32),
            jax.ShapeDtypeStruct((NC, N, C), jnp.float32),
        ],
    )(cnt, x)


def _tc_layer1(s1, y1s, dinv, W1, b1, N, C, R):
    def body(s1_ref, y1_ref, dinv_ref, w_ref, b_ref, y2_ref):
        dv = dinv_ref[...]
        z = (s1_ref[0] + s1_ref[1] + y1_ref[0]) * dv
        h = jnp.dot(z, w_ref[...], preferred_element_type=jnp.float32) + b_ref[...]
        y2 = jnp.maximum(h, 0.0) * dv
        y2_ref[0] = y2[:, :C]
        y2_ref[1] = y2[:, C:]

    return pl.pallas_call(
        body,
        grid=(N // R,),
        in_specs=[
            pl.BlockSpec((NC, R, C), lambda i: (0, i, 0)),
            pl.BlockSpec((1, R, C), lambda i: (0, i, 0)),
            pl.BlockSpec((R, 1), lambda i: (i, 0)),
            pl.BlockSpec((C, 2 * C), lambda i: (0, 0)),
            pl.BlockSpec((1, 2 * C), lambda i: (0, 0)),
        ],
        out_specs=pl.BlockSpec((NC, R, C), lambda i: (0, i, 0)),
        out_shape=jax.ShapeDtypeStruct((NC, N, C), jnp.float32),
    )(s1, y1s, dinv, W1, b1)


def _tc_layer2(sA, sB, y2s, dinv, wma, wmb, wla, wlb, bm, bl, N, C, R):
    def body(sA_ref, sB_ref, y2_ref, dinv_ref, wma_ref, wmb_ref, wla_ref,
             wlb_ref, bm_ref, bl_ref, mu_ref, ls_ref):
        dv = dinv_ref[...]
        qa = (sA_ref[0] + sB_ref[0] + y2_ref[0]) * dv
        qb = (sA_ref[1] + sB_ref[1] + y2_ref[1]) * dv
        mu = (jnp.dot(qa, wma_ref[...], preferred_element_type=jnp.float32)
              + jnp.dot(qb, wmb_ref[...], preferred_element_type=jnp.float32)
              + bm_ref[...])
        ls = (jnp.dot(qa, wla_ref[...], preferred_element_type=jnp.float32)
              + jnp.dot(qb, wlb_ref[...], preferred_element_type=jnp.float32)
              + bl_ref[...])
        mu_ref[...] = mu
        ls_ref[...] = jnp.minimum(ls, 10.0)

    blk = pl.BlockSpec((NC, R, C), lambda i: (0, i, 0))
    mat = pl.BlockSpec((C, C), lambda i: (0, 0))
    vec = pl.BlockSpec((1, C), lambda i: (0, 0))
    return pl.pallas_call(
        body,
        grid=(N // R,),
        in_specs=[
            blk, blk, blk,
            pl.BlockSpec((R, 1), lambda i: (i, 0)),
            mat, mat, mat, mat, vec, vec,
        ],
        out_specs=[
            pl.BlockSpec((R, C), lambda i: (i, 0)),
            pl.BlockSpec((R, C), lambda i: (i, 0)),
        ],
        out_shape=[
            jax.ShapeDtypeStruct((N, C), jnp.float32),
            jax.ShapeDtypeStruct((N, C), jnp.float32),
        ],
    )(sA, sB, y2s, dinv, wma, wmb, wla, wlb, bm, bl)


def kernel(x, edge_index, W1, b1, W_mu, b_mu, W_ls, b_ls):
    N, C = x.shape
    E = edge_index.shape[1]
    EH = E // 2
    NP = _pad_rows(N)
    R = 1000
    nch = EH // NS // K

    src = edge_index[0].astype(jnp.int32)
    dst = edge_index[1].astype(jnp.int32)
    shp = (NC, NS, nch, K)
    src4_l1 = src.reshape(shp)
    dst4_l1 = dst.reshape(shp)
    src4_2a = jnp.stack([src[:EH], src[EH:] + N]).reshape(shp)
    src4_2b = jnp.stack([src[EH:], src[:EH] + N]).reshape(shp)
    dst4_2a = dst4_l1
    dst4_2b = jnp.stack([dst[EH:], dst[:EH]]).reshape(shp)
    zeros_c = jnp.zeros((NP, C), jnp.float32)
    ones_t = jnp.ones((NC * N, C), jnp.float32)

    cnt = _sc_prop(ones_t, jnp.zeros(shp, jnp.int32), dst4_l1, zeros_c, NP, C)
    dinv, y1s = _tc_prep(cnt, x, N, C, R)
    s1 = _sc_prop(y1s.reshape(NC * N, C), src4_l1, dst4_l1, zeros_c, NP, C)
    y2s = _tc_layer1(s1, y1s, dinv, W1, b1.reshape(1, -1), N, C, R)
    y2f = y2s.reshape(NC * N, C)
    s2a = _sc_prop(y2f, src4_2a, dst4_2a, zeros_c, NP, C)
    s2b = _sc_prop(y2f, src4_2b, dst4_2b, zeros_c, NP, C)
    mu, ls = _tc_layer2(
        s2a, s2b, y2s, dinv, W_mu[:C], W_mu[C:], W_ls[:C], W_ls[C:],
        b_mu.reshape(1, -1), b_ls.reshape(1, -1), N, C, R)
    return (mu, ls)


# speedup vs baseline: 17.5902x; 17.5902x over previous
import functools

import jax
import jax.numpy as jnp
from jax import lax
from jax.experimental import pallas as pl
from jax.experimental.pallas import tpu as pltpu
from jax.experimental.pallas import tpu_sc as plsc

NC = 2
NS = 16
K = 80


def _mesh():
    return plsc.VectorSubcoreMesh(core_axis_name="c", subcore_axis_name="s")


def _pad_rows(n):
    return ((n + NS * 8 - 1) // (NS * 8)) * NS * 8


def _sc_prop(table, src4, dst4, zeros, NP, W):
    nch = src4.shape[2]
    rpt = NP // NS
    nh1 = ((nch + 1) // 2 + 7) // 8 * 8
    halves = ((0, nh1), (nh1, nch - nh1))

    @functools.partial(
        pl.kernel,
        out_type=jax.ShapeDtypeStruct((NC, NP, W), jnp.float32),
        mesh=_mesh(),
        scratch_types=[
            pltpu.VMEM((nh1, K), jnp.int32),
            pltpu.VMEM((nh1, K), jnp.int32),
            pltpu.VMEM((2, K, W), jnp.float32),
            pltpu.VMEM_SHARED((NP, W), jnp.float32),
            pltpu.SemaphoreType.DMA,
        ],
    )
    def body(t_hbm, src_hbm, dst_hbm, z_hbm, out_hbm,
             src_v, dst_v, rows2, acc, sem):
        cid = lax.axis_index("c")
        sid = lax.axis_index("s")
        r0 = sid * rpt
        pltpu.sync_copy(z_hbm.at[pl.ds(r0, rpt)], acc.at[pl.ds(r0, rpt)])
        plsc.subcore_barrier()

        for base, cnt in halves:
            pltpu.sync_copy(src_hbm.at[cid, sid, pl.ds(base, cnt)],
                            src_v.at[pl.ds(0, cnt)])
            pltpu.sync_copy(dst_hbm.at[cid, sid, pl.ds(base, cnt)],
                            dst_v.at[pl.ds(0, cnt)])
            pltpu.async_copy(t_hbm.at[src_v.at[0]], rows2.at[0], sem)

            def step(i, carry):
                p = lax.rem(i, 2)
                pltpu.make_async_copy(
                    t_hbm.at[src_v.at[i]], rows2.at[p], sem).wait()

                @pl.when(i + 1 < cnt)
                def _():
                    pltpu.async_copy(
                        t_hbm.at[src_v.at[i + 1]], rows2.at[1 - p], sem)

                pltpu.sync_copy(rows2.at[p], acc.at[dst_v.at[i]], add=True)
                return carry

            lax.fori_loop(0, cnt, step, 0)

        plsc.subcore_barrier()
        pltpu.sync_copy(acc.at[pl.ds(r0, rpt)], out_hbm.at[cid, pl.ds(r0, rpt)])

    return body(table, src4, dst4, zeros)


def _tc_prep(cnt, x, N, C, R):
    def body(cnt_ref, x_ref, dinv_ref, y1_ref):
        deg = cnt_ref[0][:, :1] + cnt_ref[1][:, :1] + 1.0
        dinv = lax.rsqrt(deg)
        dinv_ref[...] = dinv
        y1 = x_ref[...] * dinv
        y1_ref[0] = y1
        y1_ref[1] = y1

    return pl.pallas_call(
        body,
        grid=(N // R,),
        in_specs=[
            pl.BlockSpec((NC, R, C), lambda i: (0, i, 0)),
            pl.BlockSpec((R, C), lambda i: (i, 0)),
        ],
        out_specs=[
            pl.BlockSpec((R, 1), lambda i: (i, 0)),
            pl.BlockSpec((NC, R, C), lambda i: (0, i, 0)),
        ],
        out_shape=[
            jax.ShapeDtypeStruct((N, 1), jnp.float32),
            jax.ShapeDtypeStruct((NC, N, C), jnp.float32),
        ],
    )(cnt, x)


def _tc_layer1(s1, y1s, dinv, W1, b1, N, C, R):
    def body(s1_ref, y1_ref, dinv_ref, w_ref, b_ref, y2_ref):
        dv = dinv_ref[...]
        z = (s1_ref[0] + s1_ref[1] + y1_ref[0]) * dv
        h = jnp.dot(z, w_ref[...], preferred_element_type=jnp.float32) + b_ref[...]
        y2 = jnp.maximum(h, 0.0) * dv
        y2_ref[0] = y2[:, :C]
        y2_ref[1] = y2[:, C:]

    return pl.pallas_call(
        body,
        grid=(N // R,),
        in_specs=[
            pl.BlockSpec((NC, R, C), lambda i: (0, i, 0)),
            pl.BlockSpec((1, R, C), lambda i: (0, i, 0)),
            pl.BlockSpec((R, 1), lambda i: (i, 0)),
            pl.BlockSpec((C, 2 * C), lambda i: (0, 0)),
            pl.BlockSpec((1, 2 * C), lambda i: (0, 0)),
        ],
        out_specs=pl.BlockSpec((NC, R, C), lambda i: (0, i, 0)),
        out_shape=jax.ShapeDtypeStruct((NC, N, C), jnp.float32),
    )(s1, y1s, dinv, W1, b1)


def _tc_layer2(sA, sB, y2s, dinv, wma, wmb, wla, wlb, bm, bl, N, C, R):
    def body(sA_ref, sB_ref, y2_ref, dinv_ref, wma_ref, wmb_ref, wla_ref,
             wlb_ref, bm_ref, bl_ref, mu_ref, ls_ref):
        dv = dinv_ref[...]
        qa = (sA_ref[0] + sB_ref[0] + y2_ref[0]) * dv
        qb = (sA_ref[1] + sB_ref[1] + y2_ref[1]) * dv
        mu = (jnp.dot(qa, wma_ref[...], preferred_element_type=jnp.float32)
              + jnp.dot(qb, wmb_ref[...], preferred_element_type=jnp.float32)
              + bm_ref[...])
        ls = (jnp.dot(qa, wla_ref[...], preferred_element_type=jnp.float32)
              + jnp.dot(qb, wlb_ref[...], preferred_element_type=jnp.float32)
              + bl_ref[...])
        mu_ref[...] = mu
        ls_ref[...] = jnp.minimum(ls, 10.0)

    blk = pl.BlockSpec((NC, R, C), lambda i: (0, i, 0))
    mat = pl.BlockSpec((C, C), lambda i: (0, 0))
    vec = pl.BlockSpec((1, C), lambda i: (0, 0))
    return pl.pallas_call(
        body,
        grid=(N // R,),
        in_specs=[
            blk, blk, blk,
            pl.BlockSpec((R, 1), lambda i: (i, 0)),
            mat, mat, mat, mat, vec, vec,
        ],
        out_specs=[
            pl.BlockSpec((R, C), lambda i: (i, 0)),
            pl.BlockSpec((R, C), lambda i: (i, 0)),
        ],
        out_shape=[
            jax.ShapeDtypeStruct((N, C), jnp.float32),
            jax.ShapeDtypeStruct((N, C), jnp.float32),
        ],
    )(sA, sB, y2s, dinv, wma, wmb, wla, wlb, bm, bl)


def kernel(x, edge_index, W1, b1, W_mu, b_mu, W_ls, b_ls):
    N, C = x.shape
    E = edge_index.shape[1]
    EH = E // 2
    NP = _pad_rows(N)
    R = 1000
    nch = EH // NS // K

    src = edge_index[0].astype(jnp.int32)
    dst = edge_index[1].astype(jnp.int32)
    shp = (NC, NS, nch, K)
    src4_l1 = src.reshape(shp)
    dst4_l1 = dst.reshape(shp)
    src4_2a = jnp.stack([src[:EH], src[EH:] + N]).reshape(shp)
    src4_2b = jnp.stack([src[EH:], src[:EH] + N]).reshape(shp)
    dst4_2a = dst4_l1
    dst4_2b = jnp.stack([dst[EH:], dst[:EH]]).reshape(shp)
    zeros_c = jnp.zeros((NP, C), jnp.float32)
    ones_t = jnp.ones((NC * N, C), jnp.float32)

    src4_cnt = (jnp.arange(NC * NS * nch * K, dtype=jnp.int32) % (2 * N)
                ).reshape(shp)
    cnt = _sc_prop(ones_t, src4_cnt, dst4_l1, zeros_c, NP, C)
    dinv, y1s = _tc_prep(cnt, x, N, C, R)
    s1 = _sc_prop(y1s.reshape(NC * N, C), src4_l1, dst4_l1, zeros_c, NP, C)
    y2s = _tc_layer1(s1, y1s, dinv, W1, b1.reshape(1, -1), N, C, R)
    y2f = y2s.reshape(NC * N, C)
    s2a = _sc_prop(y2f, src4_2a, dst4_2a, zeros_c, NP, C)
    s2b = _sc_prop(y2f, src4_2b, dst4_2b, zeros_c, NP, C)
    mu, ls = _tc_layer2(
        s2a, s2b, y2s, dinv, W_mu[:C], W_mu[C:], W_ls[:C], W_ls[C:],
        b_mu.reshape(1, -1), b_ls.reshape(1, -1), N, C, R)
    return (mu, ls)
